# baseline (speedup 1.0000x reference)
import functools

import jax
import jax.numpy as jnp
from jax import lax
from jax.experimental import pallas as pl
from jax.experimental.pallas import tpu as pltpu
from jax.experimental.pallas import tpu_sc as plsc

N_NODES = 10000
N_EDGES = 320000
HID = 128

NC = 2
NS = 16
NW = NC * NS
CHUNK = 80
LANES = 16
EDGES_PER_W = N_EDGES // NW
N_PAD = 10240
ROWS_PER_SUB = N_PAD // NS

NBUF = 3
GD = 1
CHUNKS_PER_W = EDGES_PER_W // CHUNK + 1
EDGES_PER_W_X = CHUNKS_PER_W * CHUNK

BR = 2000
GRID = N_NODES // BR
GBR = 2048
GGRID = N_PAD // GBR


def _prep_body(x_ref, h_ref, wxt_ref, wht_ref, bm_ref, m_ref):
    m_ref[...] = (
        jnp.dot(x_ref[...], wxt_ref[...])
        + jnp.dot(h_ref[...], wht_ref[...])
        + bm_ref[...]
    )


def _prep(x, h, wxt, wht, bm):
    return pl.pallas_call(
        _prep_body,
        grid=(GRID,),
        in_specs=[
            pl.BlockSpec((BR, HID), lambda i: (i, 0)),
            pl.BlockSpec((BR, HID), lambda i: (i, 0)),
            pl.BlockSpec((HID, HID), lambda i: (0, 0)),
            pl.BlockSpec((HID, HID), lambda i: (0, 0)),
            pl.BlockSpec((1, HID), lambda i: (0, 0)),
        ],
        out_specs=pl.BlockSpec((BR, HID), lambda i: (i, 0)),
        out_shape=jax.ShapeDtypeStruct((N_NODES, HID), jnp.float32),
    )(x, h, wxt, wht, bm)


def _seg_body(m_hbm, pk_hbm, za_hbm, zc_hbm, sums_hbm, cnts_hbm,
              pks, idx_s, idx_d, rows, cnt, acc, semp, semg, sems):
    cid = lax.axis_index("c")
    sid = lax.axis_index("s")
    wid = cid * NS + sid

    pltpu.sync_copy(za_hbm, acc.at[pl.ds(sid * ROWS_PER_SUB, ROWS_PER_SUB)])
    pltpu.sync_copy(zc_hbm, cnt)
    plsc.subcore_barrier()

    base = wid * EDGES_PER_W_X
    ones = jnp.full((LANES,), 1.0, jnp.float32)

    def pk_start(j, b):
        off = pl.multiple_of(base + j * CHUNK, 8)
        pltpu.async_copy(pk_hbm.at[pl.ds(off, CHUNK)], pks[b], semp[b])

    def pk_wait(b):
        pltpu.make_async_copy(pk_hbm.at[pl.ds(0, CHUNK)], pks[b], semp[b]).wait()

    def unpack(b):
        for v in range(CHUNK // LANES):
            pk = pks[b][pl.ds(v * LANES, LANES)]
            dvec = lax.bitwise_and(pk, 16383)
            idx_s[b][pl.ds(v * LANES, LANES)] = lax.shift_right_logical(pk, 14)
            idx_d[b][pl.ds(v * LANES, LANES)] = dvec
            plsc.addupdate_scatter(cnt, [dvec], ones)

    def g_start(b):
        pltpu.async_copy(m_hbm.at[idx_s[b]], rows[b], semg[b])

    def g_wait(b):
        pltpu.make_async_copy(m_hbm.at[idx_s[b]], rows[b], semg[b]).wait()

    def s_start(b):
        pltpu.async_copy(rows[b], acc.at[idx_d[b]], sems[b], add=True)

    def s_wait(b):
        pltpu.make_async_copy(rows[b], acc.at[idx_d[b]], sems[b]).wait()

    def turn(t, B, with_swait=True, prep=True, with_pk=True):
        P = (B + GD) % NBUF
        g_wait(B)
        s_start(B)
        if prep:
            pk_wait(P)
        if with_swait:
            s_wait(P)
        if prep:
            unpack(P)
            g_start(P)
        if with_pk:
            pk_start(t + GD + NBUF, P)

    for k in range(NBUF):
        pk_start(k, k)
    for b in range(GD):
        pk_wait(b)
        unpack(b)
        g_start(b)
        pk_start(NBUF + b, b)

    for i in range(NBUF):
        turn(i, i, with_swait=(i >= NBUF - GD))

    def body(r):
        for i in range(NBUF):
            turn(r * NBUF + i, i)

    pl.loop(1, CHUNKS_PER_W // NBUF - 2)(body)

    last = (CHUNKS_PER_W // NBUF - 2) * NBUF
    for i in range(NBUF):
        turn(last + i, i, with_pk=(last + i + GD + NBUF < CHUNKS_PER_W))
    for i in range(NBUF):
        t = last + NBUF + i
        turn(t, i, prep=(t + GD < CHUNKS_PER_W), with_pk=False)
    for b in range(GD, NBUF):
        s_wait(b)

    plsc.subcore_barrier()

    pltpu.sync_copy(
        acc.at[pl.ds(sid * ROWS_PER_SUB, ROWS_PER_SUB)],
        sums_hbm.at[cid, pl.ds(sid * ROWS_PER_SUB, ROWS_PER_SUB)],
    )
    pltpu.sync_copy(cnt, cnts_hbm.at[cid, sid])


def _segment_accumulate(m, pk, za, zc):
    mesh = plsc.VectorSubcoreMesh(core_axis_name="c", subcore_axis_name="s")
    return pl.kernel(
        _seg_body,
        out_type=[
            jax.ShapeDtypeStruct((NC, N_PAD, HID), jnp.float32),
            jax.ShapeDtypeStruct((NC, NS, N_PAD), jnp.float32),
        ],
        mesh=mesh,
        compiler_params=pltpu.CompilerParams(
            needs_layout_passes=False, use_tc_tiling_on_sc=True),
        scratch_types=[
            [pltpu.VMEM((CHUNK,), jnp.int32) for _ in range(NBUF)],
            [pltpu.VMEM((CHUNK,), jnp.int32) for _ in range(NBUF)],
            [pltpu.VMEM((CHUNK,), jnp.int32) for _ in range(NBUF)],
            [pltpu.VMEM((CHUNK, HID), jnp.float32) for _ in range(NBUF)],
            pltpu.VMEM((N_PAD,), jnp.float32),
            pltpu.VMEM_SHARED((N_PAD, HID), jnp.float32),
            [pltpu.SemaphoreType.DMA for _ in range(NBUF)],
            [pltpu.SemaphoreType.DMA for _ in range(NBUF)],
            [pltpu.SemaphoreType.DMA for _ in range(NBUF)],
        ],
    )(m, pk, za, zc)


def _gru_body(x_ref, h_ref, part_ref, cnt_ref, wixt_ref, wict_ref, whht_ref,
              bih_ref, bhh_ref, out_ref):
    s = part_ref[0] + part_ref[1]
    n_in = jnp.sum(cnt_ref[...], axis=(0, 1))
    c = s / jnp.maximum(n_in, 1.0)[:, None]
    xb = x_ref[...]
    hb = h_ref[...]
    gi = (
        jnp.dot(xb, wixt_ref[...])
        + jnp.dot(c, wict_ref[...])
        + bih_ref[...]
    )
    gh = jnp.dot(hb, whht_ref[...]) + bhh_ref[...]
    r = jax.nn.sigmoid(gi[:, :HID] + gh[:, :HID])
    z = jax.nn.sigmoid(gi[:, HID:2 * HID] + gh[:, HID:2 * HID])
    nn_ = jnp.tanh(gi[:, 2 * HID:] + r * gh[:, 2 * HID:])
    out_ref[...] = (1.0 - z) * nn_ + z * hb


def _gru(x, h, partials, counts, wixt, wict, whht, bih, bhh):
    return pl.pallas_call(
        _gru_body,
        grid=(GGRID,),
        in_specs=[
            pl.BlockSpec((GBR, HID), lambda i: (i, 0)),
            pl.BlockSpec((GBR, HID), lambda i: (i, 0)),
            pl.BlockSpec((NC, GBR, HID), lambda i: (0, i, 0)),
            pl.BlockSpec((NC, NS, GBR), lambda i: (0, 0, i)),
            pl.BlockSpec((HID, 3 * HID), lambda i: (0, 0)),
            pl.BlockSpec((HID, 3 * HID), lambda i: (0, 0)),
            pl.BlockSpec((HID, 3 * HID), lambda i: (0, 0)),
            pl.BlockSpec((1, 3 * HID), lambda i: (0, 0)),
            pl.BlockSpec((1, 3 * HID), lambda i: (0, 0)),
        ],
        out_specs=pl.BlockSpec((GBR, HID), lambda i: (i, 0)),
        out_shape=jax.ShapeDtypeStruct((N_NODES, HID), jnp.float32),
    )(x, h, partials, counts, wixt, wict, whht, bih, bhh)


def kernel(x, h, edge_index, W_msg, b_msg, W_ih, W_hh, b_ih, b_hh):
    src = edge_index[0].astype(jnp.int32)
    dst = edge_index[1].astype(jnp.int32)
    pk = lax.bitwise_or(lax.shift_left(src, 14), dst)
    pk = jnp.concatenate(
        [pk.reshape(NW, EDGES_PER_W),
         jnp.full((NW, CHUNK), N_NODES, jnp.int32)], axis=1).reshape(-1)
    pk = jnp.pad(pk, (0, PK_PAD - NW * EDGES_PER_W_X))

    wxt = W_msg[:, :HID].T
    wht = W_msg[:, HID:].T
    bm = b_msg[None, :]
    wixt = W_ih[:, :HID].T
    wict = W_ih[:, HID:].T
    whht = W_hh.T
    bih = b_ih[None, :]
    bhh = b_hh[None, :]

    za = jnp.zeros((ROWS_PER_SUB, HID), jnp.float32)
    zc = jnp.zeros((N_PAD,), jnp.float32)

    m = _prep(x, h, wxt, wht, bm)
    partials, counts = _segment_accumulate(m, pk, za, zc)
    return _gru(x, h, partials, counts, wixt, wict, whht, bih, bhh)


# speedup vs baseline: 1.2270x; 1.2270x over previous
import functools

import jax
import jax.numpy as jnp
from jax import lax
from jax.experimental import pallas as pl
from jax.experimental.pallas import tpu as pltpu
from jax.experimental.pallas import tpu_sc as plsc

N_NODES = 10000
N_EDGES = 320000
HID = 128

NC = 2
NS = 16
NW = NC * NS
CHUNK = 80
LANES = 16
EDGES_PER_W = N_EDGES // NW
N_PAD = 10240
ROWS_PER_SUB = N_PAD // NS

NSLOT = 3
CHUNKS_PER_W = EDGES_PER_W // CHUNK + 1
EDGES_PER_W_X = CHUNKS_PER_W * CHUNK
PK_PAD = NW * EDGES_PER_W_X + 4 * CHUNK

BR = 2000
GRID = N_NODES // BR
GBR = 2048
GGRID = N_PAD // GBR


def _prep_body(x_ref, h_ref, wxt_ref, wht_ref, bm_ref, m_ref):
    m_ref[...] = (
        jnp.dot(x_ref[...], wxt_ref[...])
        + jnp.dot(h_ref[...], wht_ref[...])
        + bm_ref[...]
    )


def _prep(x, h, wxt, wht, bm):
    return pl.pallas_call(
        _prep_body,
        grid=(GRID,),
        in_specs=[
            pl.BlockSpec((BR, HID), lambda i: (i, 0)),
            pl.BlockSpec((BR, HID), lambda i: (i, 0)),
            pl.BlockSpec((HID, HID), lambda i: (0, 0)),
            pl.BlockSpec((HID, HID), lambda i: (0, 0)),
            pl.BlockSpec((1, HID), lambda i: (0, 0)),
        ],
        out_specs=pl.BlockSpec((BR, HID), lambda i: (i, 0)),
        out_shape=jax.ShapeDtypeStruct((N_NODES, HID), jnp.float32),
    )(x, h, wxt, wht, bm)


def _seg_body(m_hbm, pk_hbm, za_hbm, zc_hbm, sums_hbm, cnts_hbm,
              pks, idx_s, idx_d, rows, cnt, acc, semp, semg, sems):
    cid = lax.axis_index("c")
    sid = lax.axis_index("s")
    wid = cid * NS + sid

    pltpu.sync_copy(za_hbm, acc.at[pl.ds(sid * ROWS_PER_SUB, ROWS_PER_SUB)])
    pltpu.sync_copy(zc_hbm, cnt)
    plsc.subcore_barrier()

    base = wid * EDGES_PER_W_X
    ones = jnp.full((LANES,), 1.0, jnp.float32)

    def pk_start(j, b):
        off = pl.multiple_of(base + j * CHUNK, 8)
        pltpu.async_copy(pk_hbm.at[pl.ds(off, CHUNK)], pks[b], semp[b])

    def pk_wait(b):
        pltpu.make_async_copy(pk_hbm.at[pl.ds(0, CHUNK)], pks[b], semp[b]).wait()

    def unpack(b):
        for v in range(CHUNK // LANES):
            pk = pks[b][pl.ds(v * LANES, LANES)]
            dvec = lax.bitwise_and(pk, 16383)
            idx_s[b][pl.ds(v * LANES, LANES)] = lax.shift_right_logical(pk, 14)
            idx_d[b][pl.ds(v * LANES, LANES)] = dvec
            plsc.addupdate_scatter(cnt, [dvec], ones)

    def g_start(b):
        pltpu.async_copy(m_hbm.at[idx_s[b]], rows[b], semg[b])

    def g_wait(b):
        pltpu.make_async_copy(m_hbm.at[idx_s[b]], rows[b], semg[b]).wait()

    def scat(b):
        pltpu.sync_copy(rows[b], acc.at[idx_d[b]], add=True)

    for b in range(NSLOT):
        pk_start(b, b)
    for b in range(NSLOT):
        pk_wait(b)
        unpack(b)
        g_start(b)
        pk_start(b + NSLOT, b)

    def body(t):
        j = t * NSLOT
        for b in range(NSLOT):
            g_wait(b)
            scat(b)
            pk_wait(b)
            unpack(b)
            g_start(b)
            pk_start(j + 2 * NSLOT + b, b)

    pl.loop(0, CHUNKS_PER_W // NSLOT - 1)(body)
    for b in range(NSLOT):
        g_wait(b)
        scat(b)
        pk_wait(b)

    plsc.subcore_barrier()

    pltpu.sync_copy(
        acc.at[pl.ds(sid * ROWS_PER_SUB, ROWS_PER_SUB)],
        sums_hbm.at[cid, pl.ds(sid * ROWS_PER_SUB, ROWS_PER_SUB)],
    )
    pltpu.sync_copy(cnt, cnts_hbm.at[cid, sid])


def _segment_accumulate(m, pk, za, zc):
    mesh = plsc.VectorSubcoreMesh(core_axis_name="c", subcore_axis_name="s")
    return pl.kernel(
        _seg_body,
        out_type=[
            jax.ShapeDtypeStruct((NC, N_PAD, HID), jnp.float32),
            jax.ShapeDtypeStruct((NC, NS, N_PAD), jnp.float32),
        ],
        mesh=mesh,
        compiler_params=pltpu.CompilerParams(
            needs_layout_passes=False, use_tc_tiling_on_sc=True),
        scratch_types=[
            [pltpu.VMEM((CHUNK,), jnp.int32) for _ in range(NSLOT)],
            [pltpu.VMEM((CHUNK,), jnp.int32) for _ in range(NSLOT)],
            [pltpu.VMEM((CHUNK,), jnp.int32) for _ in range(NSLOT)],
            [pltpu.VMEM((CHUNK, HID), jnp.float32) for _ in range(NSLOT)],
            pltpu.VMEM((N_PAD,), jnp.float32),
            pltpu.VMEM_SHARED((N_PAD, HID), jnp.float32),
            [pltpu.SemaphoreType.DMA for _ in range(NSLOT)],
            [pltpu.SemaphoreType.DMA for _ in range(NSLOT)],
            [pltpu.SemaphoreType.DMA for _ in range(NSLOT)],
        ],
    )(m, pk, za, zc)


def _gru_body(x_ref, h_ref, part_ref, cnt_ref, wixt_ref, wict_ref, whht_ref,
              bih_ref, bhh_ref, out_ref):
    s = part_ref[0] + part_ref[1]
    n_in = jnp.sum(cnt_ref[...], axis=(0, 1))
    c = s / jnp.maximum(n_in, 1.0)[:, None]
    xb = x_ref[...]
    hb = h_ref[...]
    gi = (
        jnp.dot(xb, wixt_ref[...])
        + jnp.dot(c, wict_ref[...])
        + bih_ref[...]
    )
    gh = jnp.dot(hb, whht_ref[...]) + bhh_ref[...]
    r = jax.nn.sigmoid(gi[:, :HID] + gh[:, :HID])
    z = jax.nn.sigmoid(gi[:, HID:2 * HID] + gh[:, HID:2 * HID])
    nn_ = jnp.tanh(gi[:, 2 * HID:] + r * gh[:, 2 * HID:])
    out_ref[...] = (1.0 - z) * nn_ + z * hb


def _gru(x, h, partials, counts, wixt, wict, whht, bih, bhh):
    return pl.pallas_call(
        _gru_body,
        grid=(GGRID,),
        in_specs=[
            pl.BlockSpec((GBR, HID), lambda i: (i, 0)),
            pl.BlockSpec((GBR, HID), lambda i: (i, 0)),
            pl.BlockSpec((NC, GBR, HID), lambda i: (0, i, 0)),
            pl.BlockSpec((NC, NS, GBR), lambda i: (0, 0, i)),
            pl.BlockSpec((HID, 3 * HID), lambda i: (0, 0)),
            pl.BlockSpec((HID, 3 * HID), lambda i: (0, 0)),
            pl.BlockSpec((HID, 3 * HID), lambda i: (0, 0)),
            pl.BlockSpec((1, 3 * HID), lambda i: (0, 0)),
            pl.BlockSpec((1, 3 * HID), lambda i: (0, 0)),
        ],
        out_specs=pl.BlockSpec((GBR, HID), lambda i: (i, 0)),
        out_shape=jax.ShapeDtypeStruct((N_NODES, HID), jnp.float32),
    )(x, h, partials, counts, wixt, wict, whht, bih, bhh)


def kernel(x, h, edge_index, W_msg, b_msg, W_ih, W_hh, b_ih, b_hh):
    src = edge_index[0].astype(jnp.int32)
    dst = edge_index[1].astype(jnp.int32)
    pk = lax.bitwise_or(lax.shift_left(src, 14), dst)
    pk = jnp.concatenate(
        [pk.reshape(NW, EDGES_PER_W),
         jnp.full((NW, CHUNK), N_NODES, jnp.int32)], axis=1).reshape(-1)
    pk = jnp.pad(pk, (0, PK_PAD - NW * EDGES_PER_W_X))

    wxt = W_msg[:, :HID].T
    wht = W_msg[:, HID:].T
    bm = b_msg[None, :]
    wixt = W_ih[:, :HID].T
    wict = W_ih[:, HID:].T
    whht = W_hh.T
    bih = b_ih[None, :]
    bhh = b_hh[None, :]

    za = jnp.zeros((ROWS_PER_SUB, HID), jnp.float32)
    zc = jnp.zeros((N_PAD,), jnp.float32)

    m = _prep(x, h, wxt, wht, bm)
    partials, counts = _segment_accumulate(m, pk, za, zc)
    return _gru(x, h, partials, counts, wixt, wict, whht, bih, bhh)
